# baseline (speedup 1.0000x reference)
import functools

import numpy as np
import jax
import jax.numpy as jnp
from jax import lax
from jax.experimental import pallas as pl
from jax.experimental.pallas import tpu as pltpu
from jax.experimental.pallas import tpu_sc as plsc

_NUM_LEVELS = 16
_LOG2_T = 14
_T = 1 << _LOG2_T
_F = 8
_N = 131072
_GROWTH = np.exp((np.log(1024.0) - np.log(16.0)) / (_NUM_LEVELS - 1))
_RES = np.floor(16.0 * _GROWTH ** np.arange(_NUM_LEVELS)).astype(np.float32)

_K2 = np.uint32(2654435761).astype(np.int32)
_K3 = np.int32(805459861)
_MASK = np.int32(_T - 1)

_NC, _NS = 2, 16
_NW = _NC * _NS
_PW = _N // _NW
_P = 128
_NCHUNK = _PW // _P
_G = _P // 16
_H = 4 * _P
_NSPM = 2
_LSPM = _NUM_LEVELS - _NSPM
_SPM_ROWS = _NSPM * _T
_CP = _SPM_ROWS // _NS


def _body(xyz, table, resb, out, xyz_v, res_v, wh0_v, wh1_v, ws0_v, ws1_v,
          idxh0_v, idxh1_v, idxs0_v, idxs1_v, rh0_v, rh1_v, rs0_v, rs1_v,
          out_v, spm_v, sem_in, semh0a, semh0b, semh1a, semh1b, sems0a,
          sems0b, sems1a, sems1b, sem_out):
    wid = lax.axis_index("s") * jnp.int32(_NC) + lax.axis_index("c")
    sid = lax.axis_index("s")
    lanes = lax.iota(jnp.int32, 16)

    pltpu.async_copy(resb, res_v, sem_in).wait()

    cbase = sid * jnp.int32(_CP)
    pltpu.async_copy(
        table.at[pl.ds(jnp.int32(_LSPM * _T) + cbase, _CP)],
        spm_v.at[pl.ds(cbase, _CP)], sem_in).wait()
    plsc.subcore_barrier()

    def hash_pass(l, loff, idx_v, w_v):
        res_vec = res_v[pl.ds(l * jnp.int32(16), 16)]

        def hash_body(g, carry):
            o = g * jnp.int32(16)
            x = xyz_v[0, pl.ds(o, 16)]
            y = xyz_v[1, pl.ds(o, 16)]
            z = xyz_v[2, pl.ds(o, 16)]
            sx = x * res_vec
            sy = y * res_vec
            sz = z * res_vec
            fx = sx.astype(jnp.int32)
            fy = sy.astype(jnp.int32)
            fz = sz.astype(jnp.int32)
            w_v[0, pl.ds(o, 16)] = sx - fx.astype(jnp.float32)
            w_v[1, pl.ds(o, 16)] = sy - fy.astype(jnp.float32)
            w_v[2, pl.ds(o, 16)] = sz - fz.astype(jnp.float32)
            b0 = fy * _K2
            c0 = fz * _K3
            ax = (fx, fx + jnp.int32(1))
            by = (b0, b0 + _K2)
            cz = (c0, c0 + _K3)
            for zb in range(2):
                for yb in range(2):
                    for xb in range(2):
                        h = ((ax[xb] ^ by[yb] ^ cz[zb]) & _MASK) + loff
                        slot = xb + 2 * yb + 4 * zb
                        idx_v[pl.ds(jnp.int32(slot * _P) + o, 16)] = h
            return carry

        lax.fori_loop(jnp.int32(0), jnp.int32(_G), hash_body, jnp.int32(0))

    def interp_pass(l, rows_v, w_v):
        lf = l * jnp.int32(_F)
        cols = [jnp.full((16,), f, jnp.int32) for f in range(_F)]

        @plsc.parallel_loop(jnp.int32(0), jnp.int32(_G), jnp.int32(1),
                            unroll=2)
        def interp_body(g):
            o = g * jnp.int32(16)
            wx = w_v[0, pl.ds(o, 16)]
            wy = w_v[1, pl.ds(o, 16)]
            wz = w_v[2, pl.ds(o, 16)]
            ux = 1.0 - wx
            uy = 1.0 - wy
            uz = 1.0 - wz
            p00 = ux * uy
            p10 = wx * uy
            p01 = ux * wy
            p11 = wx * wy
            w8 = [p00 * uz, p10 * uz, p01 * uz, p11 * uz,
                  p00 * wz, p10 * wz, p01 * wz, p11 * wz]
            r = o + lanes
            rows = [r + jnp.int32(s * _P) for s in range(8)]
            accs = [None] * _F
            for s in range(8):
                for f in range(_F):
                    v = plsc.load_gather(rows_v, [rows[s], cols[f]])
                    t = v * w8[s]
                    accs[f] = t if accs[f] is None else accs[f] + t
            for f in range(_F):
                ocol = jnp.broadcast_to(lf + jnp.int32(f), (16,))
                plsc.store_scatter(out_v, [r, ocol], accs[f])

    def start_gather(src, idx_v, rows_v, sa, sb):
        pltpu.async_copy(src.at[idx_v.at[pl.ds(0, _H)]],
                         rows_v.at[pl.ds(0, _H)], sa)
        pltpu.async_copy(src.at[idx_v.at[pl.ds(_H, _H)]],
                         rows_v.at[pl.ds(_H, _H)], sb)

    def wait_gather(src, idx_v, rows_v, sa, sb):
        pltpu.make_async_copy(src.at[idx_v.at[pl.ds(0, _H)]],
                              rows_v.at[pl.ds(0, _H)], sa).wait()
        pltpu.make_async_copy(src.at[idx_v.at[pl.ds(_H, _H)]],
                              rows_v.at[pl.ds(_H, _H)], sb).wait()

    def chunk_body(ci, carry):
        base = wid * jnp.int32(_PW) + ci * jnp.int32(_P)
        pltpu.async_copy(xyz.at[:, pl.ds(base, _P)], xyz_v, sem_in).wait()

        hash_pass(jnp.int32(_LSPM), jnp.int32(0), idxs0_v, ws0_v)
        start_gather(spm_v, idxs0_v, rs0_v, sems0a, sems0b)
        hash_pass(jnp.int32(_LSPM + 1), jnp.int32(_T), idxs1_v, ws1_v)
        start_gather(spm_v, idxs1_v, rs1_v, sems1a, sems1b)

        hash_pass(jnp.int32(0), jnp.int32(0), idxh0_v, wh0_v)
        start_gather(table, idxh0_v, rh0_v, semh0a, semh0b)

        def dbl_body(k, carry2):
            j0 = k * jnp.int32(2)
            j1 = j0 + jnp.int32(1)
            hash_pass(j1, j1 * jnp.int32(_T), idxh1_v, wh1_v)
            start_gather(table, idxh1_v, rh1_v, semh1a, semh1b)

            wait_gather(table, idxh0_v, rh0_v, semh0a, semh0b)
            interp_pass(j0, rh0_v, wh0_v)

            @pl.when(k < jnp.int32(_LSPM // 2 - 1))
            def _prefetch_next():
                j2 = j0 + jnp.int32(2)
                hash_pass(j2, j2 * jnp.int32(_T), idxh0_v, wh0_v)
                start_gather(table, idxh0_v, rh0_v, semh0a, semh0b)

            wait_gather(table, idxh1_v, rh1_v, semh1a, semh1b)
            interp_pass(j1, rh1_v, wh1_v)
            return carry2

        lax.fori_loop(jnp.int32(0), jnp.int32(_LSPM // 2), dbl_body,
                      jnp.int32(0))

        wait_gather(spm_v, idxs0_v, rs0_v, sems0a, sems0b)
        interp_pass(jnp.int32(_LSPM), rs0_v, ws0_v)
        wait_gather(spm_v, idxs1_v, rs1_v, sems1a, sems1b)
        interp_pass(jnp.int32(_LSPM + 1), rs1_v, ws1_v)
        pltpu.async_copy(out_v, out.at[pl.ds(base, _P)], sem_out).wait()
        return carry

    lax.fori_loop(jnp.int32(0), jnp.int32(_NCHUNK), chunk_body, jnp.int32(0))


_hash_enc = functools.partial(
    pl.kernel,
    out_type=jax.ShapeDtypeStruct((_N, _NUM_LEVELS * _F), jnp.float32),
    mesh=plsc.VectorSubcoreMesh(core_axis_name="c", subcore_axis_name="s"),
    scratch_types=[
        pltpu.VMEM((3, _P), jnp.float32),
        pltpu.VMEM((16 * 16,), jnp.float32),
        pltpu.VMEM((3, _P), jnp.float32),
        pltpu.VMEM((3, _P), jnp.float32),
        pltpu.VMEM((3, _P), jnp.float32),
        pltpu.VMEM((3, _P), jnp.float32),
        pltpu.VMEM((8 * _P,), jnp.int32),
        pltpu.VMEM((8 * _P,), jnp.int32),
        pltpu.VMEM((8 * _P,), jnp.int32),
        pltpu.VMEM((8 * _P,), jnp.int32),
        pltpu.VMEM((8 * _P, _F), jnp.float32),
        pltpu.VMEM((8 * _P, _F), jnp.float32),
        pltpu.VMEM((8 * _P, _F), jnp.float32),
        pltpu.VMEM((8 * _P, _F), jnp.float32),
        pltpu.VMEM((_P, _NUM_LEVELS * _F), jnp.float32),
        pltpu.VMEM_SHARED((_SPM_ROWS, _F), jnp.float32),
        pltpu.SemaphoreType.DMA,
        pltpu.SemaphoreType.DMA,
        pltpu.SemaphoreType.DMA,
        pltpu.SemaphoreType.DMA,
        pltpu.SemaphoreType.DMA,
        pltpu.SemaphoreType.DMA,
        pltpu.SemaphoreType.DMA,
        pltpu.SemaphoreType.DMA,
        pltpu.SemaphoreType.DMA,
        pltpu.SemaphoreType.DMA,
    ],
    compiler_params=pltpu.CompilerParams(
        needs_layout_passes=False, use_tc_tiling_on_sc=False),
)(_body)


def kernel(inp_points, hash_table):
    xyz = inp_points.T
    resb = jnp.asarray(np.repeat(_RES, 16))
    return _hash_enc(xyz, hash_table, resb)


# speedup vs baseline: 1.0659x; 1.0659x over previous
import functools

import numpy as np
import jax
import jax.numpy as jnp
from jax import lax
from jax.experimental import pallas as pl
from jax.experimental.pallas import tpu as pltpu
from jax.experimental.pallas import tpu_sc as plsc

_NUM_LEVELS = 16
_LOG2_T = 14
_T = 1 << _LOG2_T
_F = 8
_N = 131072
_GROWTH = np.exp((np.log(1024.0) - np.log(16.0)) / (_NUM_LEVELS - 1))
_RES = np.floor(16.0 * _GROWTH ** np.arange(_NUM_LEVELS)).astype(np.float32)

_K2 = np.uint32(2654435761).astype(np.int32)
_K3 = np.int32(805459861)
_MASK = np.int32(_T - 1)

_NC, _NS = 2, 16
_NW = _NC * _NS
_PW = _N // _NW
_P = 256
_NCHUNK = _PW // _P
_G = _P // 16
_H = 4 * _P
_NSPM = 1
_LSPM = _NUM_LEVELS - _NSPM
_SPM_ROWS = _NSPM * _T
_CP = _SPM_ROWS // _NS


def _body(xyz, table, resb, out, xyz_v, res_v, wh0_v, wh1_v, ws0_v,
          idxh0_v, idxh1_v, idxs0_v, rh0_v, rh1_v, rs0_v,
          out_v, spm_v, sem_in, semh0a, semh0b, semh1a, semh1b, sems0a,
          sems0b, sem_out):
    wid = lax.axis_index("s") * jnp.int32(_NC) + lax.axis_index("c")
    sid = lax.axis_index("s")
    lanes = lax.iota(jnp.int32, 16)

    pltpu.async_copy(resb, res_v, sem_in).wait()

    cbase = sid * jnp.int32(_CP)
    pltpu.async_copy(
        table.at[pl.ds(jnp.int32(_LSPM * _T) + cbase, _CP)],
        spm_v.at[pl.ds(cbase, _CP)], sem_in).wait()
    plsc.subcore_barrier()

    def hash_pass(l, loff, idx_v, w_v):
        res_vec = res_v[pl.ds(l * jnp.int32(16), 16)]

        def hash_body(g, carry):
            o = g * jnp.int32(16)
            x = xyz_v[0, pl.ds(o, 16)]
            y = xyz_v[1, pl.ds(o, 16)]
            z = xyz_v[2, pl.ds(o, 16)]
            sx = x * res_vec
            sy = y * res_vec
            sz = z * res_vec
            fx = sx.astype(jnp.int32)
            fy = sy.astype(jnp.int32)
            fz = sz.astype(jnp.int32)
            w_v[0, pl.ds(o, 16)] = sx - fx.astype(jnp.float32)
            w_v[1, pl.ds(o, 16)] = sy - fy.astype(jnp.float32)
            w_v[2, pl.ds(o, 16)] = sz - fz.astype(jnp.float32)
            b0 = fy * _K2
            c0 = fz * _K3
            ax = (fx, fx + jnp.int32(1))
            by = (b0, b0 + _K2)
            cz = (c0, c0 + _K3)
            for zb in range(2):
                for yb in range(2):
                    for xb in range(2):
                        h = ((ax[xb] ^ by[yb] ^ cz[zb]) & _MASK) + loff
                        slot = xb + 2 * yb + 4 * zb
                        idx_v[pl.ds(jnp.int32(slot * _P) + o, 16)] = h
            return carry

        lax.fori_loop(jnp.int32(0), jnp.int32(_G), hash_body, jnp.int32(0))

    def interp_pass(l, rows_v, w_v):
        lf = l * jnp.int32(_F)
        cols = [jnp.full((16,), f, jnp.int32) for f in range(_F)]

        @plsc.parallel_loop(jnp.int32(0), jnp.int32(_G), jnp.int32(1),
                            unroll=2)
        def interp_body(g):
            o = g * jnp.int32(16)
            wx = w_v[0, pl.ds(o, 16)]
            wy = w_v[1, pl.ds(o, 16)]
            wz = w_v[2, pl.ds(o, 16)]
            ux = 1.0 - wx
            uy = 1.0 - wy
            uz = 1.0 - wz
            p00 = ux * uy
            p10 = wx * uy
            p01 = ux * wy
            p11 = wx * wy
            w8 = [p00 * uz, p10 * uz, p01 * uz, p11 * uz,
                  p00 * wz, p10 * wz, p01 * wz, p11 * wz]
            r = o + lanes
            rows = [r + jnp.int32(s * _P) for s in range(8)]
            accs = [None] * _F
            for s in range(8):
                for f in range(_F):
                    v = plsc.load_gather(rows_v, [rows[s], cols[f]])
                    t = v * w8[s]
                    accs[f] = t if accs[f] is None else accs[f] + t
            for f in range(_F):
                ocol = jnp.broadcast_to(lf + jnp.int32(f), (16,))
                plsc.store_scatter(out_v, [r, ocol], accs[f])

    def start_gather(src, idx_v, rows_v, sa, sb):
        pltpu.async_copy(src.at[idx_v.at[pl.ds(0, _H)]],
                         rows_v.at[pl.ds(0, _H)], sa)
        pltpu.async_copy(src.at[idx_v.at[pl.ds(_H, _H)]],
                         rows_v.at[pl.ds(_H, _H)], sb)

    def wait_gather(src, idx_v, rows_v, sa, sb):
        pltpu.make_async_copy(src.at[idx_v.at[pl.ds(0, _H)]],
                              rows_v.at[pl.ds(0, _H)], sa).wait()
        pltpu.make_async_copy(src.at[idx_v.at[pl.ds(_H, _H)]],
                              rows_v.at[pl.ds(_H, _H)], sb).wait()

    def chunk_body(ci, carry):
        base = wid * jnp.int32(_PW) + ci * jnp.int32(_P)
        pltpu.async_copy(xyz.at[:, pl.ds(base, _P)], xyz_v, sem_in).wait()

        hash_pass(jnp.int32(15), jnp.int32(0), idxs0_v, ws0_v)
        start_gather(spm_v, idxs0_v, rs0_v, sems0a, sems0b)

        hash_pass(jnp.int32(0), jnp.int32(0), idxh0_v, wh0_v)
        start_gather(table, idxh0_v, rh0_v, semh0a, semh0b)

        def dbl_body(k, carry2):
            j0 = k * jnp.int32(2)
            j1 = j0 + jnp.int32(1)
            hash_pass(j1, j1 * jnp.int32(_T), idxh1_v, wh1_v)
            start_gather(table, idxh1_v, rh1_v, semh1a, semh1b)

            wait_gather(table, idxh0_v, rh0_v, semh0a, semh0b)
            interp_pass(j0, rh0_v, wh0_v)

            j2 = j0 + jnp.int32(2)
            hash_pass(j2, j2 * jnp.int32(_T), idxh0_v, wh0_v)
            start_gather(table, idxh0_v, rh0_v, semh0a, semh0b)

            wait_gather(table, idxh1_v, rh1_v, semh1a, semh1b)
            interp_pass(j1, rh1_v, wh1_v)
            return carry2

        lax.fori_loop(jnp.int32(0), jnp.int32(7), dbl_body, jnp.int32(0))

        wait_gather(table, idxh0_v, rh0_v, semh0a, semh0b)
        interp_pass(jnp.int32(14), rh0_v, wh0_v)
        wait_gather(spm_v, idxs0_v, rs0_v, sems0a, sems0b)
        interp_pass(jnp.int32(15), rs0_v, ws0_v)
        pltpu.async_copy(out_v, out.at[pl.ds(base, _P)], sem_out).wait()
        return carry

    lax.fori_loop(jnp.int32(0), jnp.int32(_NCHUNK), chunk_body, jnp.int32(0))


_hash_enc = functools.partial(
    pl.kernel,
    out_type=jax.ShapeDtypeStruct((_N, _NUM_LEVELS * _F), jnp.float32),
    mesh=plsc.VectorSubcoreMesh(core_axis_name="c", subcore_axis_name="s"),
    scratch_types=[
        pltpu.VMEM((3, _P), jnp.float32),
        pltpu.VMEM((16 * 16,), jnp.float32),
        pltpu.VMEM((3, _P), jnp.float32),
        pltpu.VMEM((3, _P), jnp.float32),
        pltpu.VMEM((3, _P), jnp.float32),
        pltpu.VMEM((8 * _P,), jnp.int32),
        pltpu.VMEM((8 * _P,), jnp.int32),
        pltpu.VMEM((8 * _P,), jnp.int32),
        pltpu.VMEM((8 * _P, _F), jnp.float32),
        pltpu.VMEM((8 * _P, _F), jnp.float32),
        pltpu.VMEM((8 * _P, _F), jnp.float32),
        pltpu.VMEM((_P, _NUM_LEVELS * _F), jnp.float32),
        pltpu.VMEM_SHARED((_SPM_ROWS, _F), jnp.float32),
        pltpu.SemaphoreType.DMA,
        pltpu.SemaphoreType.DMA,
        pltpu.SemaphoreType.DMA,
        pltpu.SemaphoreType.DMA,
        pltpu.SemaphoreType.DMA,
        pltpu.SemaphoreType.DMA,
        pltpu.SemaphoreType.DMA,
        pltpu.SemaphoreType.DMA,
    ],
    compiler_params=pltpu.CompilerParams(
        needs_layout_passes=False, use_tc_tiling_on_sc=False),
)(_body)


def kernel(inp_points, hash_table):
    xyz = inp_points.T
    resb = jnp.asarray(np.repeat(_RES, 16))
    return _hash_enc(xyz, hash_table, resb)


# speedup vs baseline: 1.1839x; 1.1107x over previous
import functools

import numpy as np
import jax
import jax.numpy as jnp
from jax import lax
from jax.experimental import pallas as pl
from jax.experimental.pallas import tpu as pltpu
from jax.experimental.pallas import tpu_sc as plsc

_NUM_LEVELS = 16
_LOG2_T = 14
_T = 1 << _LOG2_T
_F = 8
_N = 131072
_GROWTH = np.exp((np.log(1024.0) - np.log(16.0)) / (_NUM_LEVELS - 1))
_RES = np.floor(16.0 * _GROWTH ** np.arange(_NUM_LEVELS)).astype(np.float32)

_K2 = np.uint32(2654435761).astype(np.int32)
_K3 = np.int32(805459861)
_MASK = np.int32(_T - 1)

_NC, _NS = 2, 16
_NW = _NC * _NS
_PW = _N // _NW
_P = 256
_NCHUNK = _PW // _P
_G = _P // 16
_H = 4 * _P
_NSPM = 1
_LSPM = _NUM_LEVELS - _NSPM
_SPM_ROWS = _NSPM * _T
_CP = _SPM_ROWS // _NS


def _body(xyz, table, resb, out, xyz_v, res_v, wh0_v, wh1_v, ws0_v,
          idxh0_v, idxh1_v, idxs0_v, rh0_v, rh1_v, rs0_v,
          out_v, spm_v, sem_in, semh0a, semh0b, semh1a, semh1b, sems0a,
          sems0b, sem_out):
    wid = lax.axis_index("s") * jnp.int32(_NC) + lax.axis_index("c")
    sid = lax.axis_index("s")
    lanes = lax.iota(jnp.int32, 16)

    pltpu.async_copy(resb, res_v, sem_in).wait()

    cbase = sid * jnp.int32(_CP)
    pltpu.async_copy(
        table.at[pl.ds(jnp.int32(_LSPM * _T) + cbase, _CP)],
        spm_v.at[pl.ds(cbase, _CP)], sem_in).wait()
    plsc.subcore_barrier()

    def hash_pass(l, loff, idx_v, w_v):
        res_vec = res_v[pl.ds(l * jnp.int32(16), 16)]

        @plsc.parallel_loop(jnp.int32(0), jnp.int32(_G), jnp.int32(1),
                            unroll=2)
        def hash_body(g):
            o = g * jnp.int32(16)
            x = xyz_v[0, pl.ds(o, 16)]
            y = xyz_v[1, pl.ds(o, 16)]
            z = xyz_v[2, pl.ds(o, 16)]
            sx = x * res_vec
            sy = y * res_vec
            sz = z * res_vec
            fx = sx.astype(jnp.int32)
            fy = sy.astype(jnp.int32)
            fz = sz.astype(jnp.int32)
            w_v[0, pl.ds(o, 16)] = sx - fx.astype(jnp.float32)
            w_v[1, pl.ds(o, 16)] = sy - fy.astype(jnp.float32)
            w_v[2, pl.ds(o, 16)] = sz - fz.astype(jnp.float32)
            b0 = fy * _K2
            c0 = fz * _K3
            ax = (fx, fx + jnp.int32(1))
            by = (b0, b0 + _K2)
            cz = (c0, c0 + _K3)
            for zb in range(2):
                for yb in range(2):
                    for xb in range(2):
                        h = ((ax[xb] ^ by[yb] ^ cz[zb]) & _MASK) + loff
                        slot = xb + 2 * yb + 4 * zb
                        idx_v[pl.ds(jnp.int32(slot * _P) + o, 16)] = h


    def interp_pass(l, rows_v, w_v):
        lf = l * jnp.int32(_F)
        cols = [jnp.full((16,), f, jnp.int32) for f in range(_F)]

        @plsc.parallel_loop(jnp.int32(0), jnp.int32(_G), jnp.int32(1),
                            unroll=4)
        def interp_body(g):
            o = g * jnp.int32(16)
            wx = w_v[0, pl.ds(o, 16)]
            wy = w_v[1, pl.ds(o, 16)]
            wz = w_v[2, pl.ds(o, 16)]
            ux = 1.0 - wx
            uy = 1.0 - wy
            uz = 1.0 - wz
            p00 = ux * uy
            p10 = wx * uy
            p01 = ux * wy
            p11 = wx * wy
            w8 = [p00 * uz, p10 * uz, p01 * uz, p11 * uz,
                  p00 * wz, p10 * wz, p01 * wz, p11 * wz]
            r = o + lanes
            rows = [r + jnp.int32(s * _P) for s in range(8)]
            accs = [None] * _F
            for s in range(8):
                for f in range(_F):
                    v = plsc.load_gather(rows_v, [rows[s], cols[f]])
                    t = v * w8[s]
                    accs[f] = t if accs[f] is None else accs[f] + t
            for f in range(_F):
                ocol = jnp.broadcast_to(lf + jnp.int32(f), (16,))
                plsc.store_scatter(out_v, [r, ocol], accs[f])

    def start_gather(src, idx_v, rows_v, sa, sb):
        pltpu.async_copy(src.at[idx_v.at[pl.ds(0, _H)]],
                         rows_v.at[pl.ds(0, _H)], sa)
        pltpu.async_copy(src.at[idx_v.at[pl.ds(_H, _H)]],
                         rows_v.at[pl.ds(_H, _H)], sb)

    def wait_gather(src, idx_v, rows_v, sa, sb):
        pltpu.make_async_copy(src.at[idx_v.at[pl.ds(0, _H)]],
                              rows_v.at[pl.ds(0, _H)], sa).wait()
        pltpu.make_async_copy(src.at[idx_v.at[pl.ds(_H, _H)]],
                              rows_v.at[pl.ds(_H, _H)], sb).wait()

    def chunk_body(ci, carry):
        base = wid * jnp.int32(_PW) + ci * jnp.int32(_P)
        pltpu.async_copy(xyz.at[:, pl.ds(base, _P)], xyz_v, sem_in).wait()

        hash_pass(jnp.int32(15), jnp.int32(0), idxs0_v, ws0_v)
        start_gather(spm_v, idxs0_v, rs0_v, sems0a, sems0b)

        hash_pass(jnp.int32(0), jnp.int32(0), idxh0_v, wh0_v)
        start_gather(table, idxh0_v, rh0_v, semh0a, semh0b)

        def dbl_body(k, carry2):
            j0 = k * jnp.int32(2)
            j1 = j0 + jnp.int32(1)
            hash_pass(j1, j1 * jnp.int32(_T), idxh1_v, wh1_v)
            start_gather(table, idxh1_v, rh1_v, semh1a, semh1b)

            wait_gather(table, idxh0_v, rh0_v, semh0a, semh0b)
            interp_pass(j0, rh0_v, wh0_v)

            j2 = j0 + jnp.int32(2)
            hash_pass(j2, j2 * jnp.int32(_T), idxh0_v, wh0_v)
            start_gather(table, idxh0_v, rh0_v, semh0a, semh0b)

            wait_gather(table, idxh1_v, rh1_v, semh1a, semh1b)
            interp_pass(j1, rh1_v, wh1_v)
            return carry2

        lax.fori_loop(jnp.int32(0), jnp.int32(7), dbl_body, jnp.int32(0))

        wait_gather(table, idxh0_v, rh0_v, semh0a, semh0b)
        interp_pass(jnp.int32(14), rh0_v, wh0_v)
        wait_gather(spm_v, idxs0_v, rs0_v, sems0a, sems0b)
        interp_pass(jnp.int32(15), rs0_v, ws0_v)
        pltpu.async_copy(out_v, out.at[pl.ds(base, _P)], sem_out).wait()
        return carry

    lax.fori_loop(jnp.int32(0), jnp.int32(_NCHUNK), chunk_body, jnp.int32(0))


_hash_enc = functools.partial(
    pl.kernel,
    out_type=jax.ShapeDtypeStruct((_N, _NUM_LEVELS * _F), jnp.float32),
    mesh=plsc.VectorSubcoreMesh(core_axis_name="c", subcore_axis_name="s"),
    scratch_types=[
        pltpu.VMEM((3, _P), jnp.float32),
        pltpu.VMEM((16 * 16,), jnp.float32),
        pltpu.VMEM((3, _P), jnp.float32),
        pltpu.VMEM((3, _P), jnp.float32),
        pltpu.VMEM((3, _P), jnp.float32),
        pltpu.VMEM((8 * _P,), jnp.int32),
        pltpu.VMEM((8 * _P,), jnp.int32),
        pltpu.VMEM((8 * _P,), jnp.int32),
        pltpu.VMEM((8 * _P, _F), jnp.float32),
        pltpu.VMEM((8 * _P, _F), jnp.float32),
        pltpu.VMEM((8 * _P, _F), jnp.float32),
        pltpu.VMEM((_P, _NUM_LEVELS * _F), jnp.float32),
        pltpu.VMEM_SHARED((_SPM_ROWS, _F), jnp.float32),
        pltpu.SemaphoreType.DMA,
        pltpu.SemaphoreType.DMA,
        pltpu.SemaphoreType.DMA,
        pltpu.SemaphoreType.DMA,
        pltpu.SemaphoreType.DMA,
        pltpu.SemaphoreType.DMA,
        pltpu.SemaphoreType.DMA,
        pltpu.SemaphoreType.DMA,
    ],
    compiler_params=pltpu.CompilerParams(
        needs_layout_passes=False, use_tc_tiling_on_sc=False),
)(_body)


def kernel(inp_points, hash_table):
    xyz = inp_points.T
    resb = jnp.asarray(np.repeat(_RES, 16))
    return _hash_enc(xyz, hash_table, resb)


# speedup vs baseline: 1.1994x; 1.0131x over previous
import functools

import numpy as np
import jax
import jax.numpy as jnp
from jax import lax
from jax.experimental import pallas as pl
from jax.experimental.pallas import tpu as pltpu
from jax.experimental.pallas import tpu_sc as plsc

_NUM_LEVELS = 16
_LOG2_T = 14
_T = 1 << _LOG2_T
_F = 8
_N = 131072
_GROWTH = np.exp((np.log(1024.0) - np.log(16.0)) / (_NUM_LEVELS - 1))
_RES = np.floor(16.0 * _GROWTH ** np.arange(_NUM_LEVELS)).astype(np.float32)

_K2 = np.uint32(2654435761).astype(np.int32)
_K3 = np.int32(805459861)
_MASK = np.int32(_T - 1)

_NC, _NS = 2, 16
_NW = _NC * _NS
_PW = _N // _NW
_P = 256
_NCHUNK = _PW // _P
_G = _P // 16
_H = 4 * _P
_NSPM = 2
_LSPM = _NUM_LEVELS - _NSPM
_SPM_ROWS = _NSPM * _T
_CP = _SPM_ROWS // _NS


def _body(xyz, table, resb, out, xyz_v, res_v, wh0_v, wh1_v, ws0_v,
          idxh0_v, idxh1_v, idxs0_v, rh0_v, rh1_v, rs0_v,
          out_v, spm_v, sem_in, semh0a, semh0b, semh1a, semh1b, sems0a,
          sems0b, sem_out):
    wid = lax.axis_index("s") * jnp.int32(_NC) + lax.axis_index("c")
    sid = lax.axis_index("s")
    lanes = lax.iota(jnp.int32, 16)

    pltpu.async_copy(resb, res_v, sem_in).wait()

    cbase = sid * jnp.int32(_CP)
    pltpu.async_copy(
        table.at[pl.ds(jnp.int32(_LSPM * _T) + cbase, _CP)],
        spm_v.at[pl.ds(cbase, _CP)], sem_in).wait()
    plsc.subcore_barrier()

    def hash_pass(l, loff, idx_v, w_v):
        res_vec = res_v[pl.ds(l * jnp.int32(16), 16)]

        @plsc.parallel_loop(jnp.int32(0), jnp.int32(_G), jnp.int32(1),
                            unroll=2)
        def hash_body(g):
            o = g * jnp.int32(16)
            x = xyz_v[0, pl.ds(o, 16)]
            y = xyz_v[1, pl.ds(o, 16)]
            z = xyz_v[2, pl.ds(o, 16)]
            sx = x * res_vec
            sy = y * res_vec
            sz = z * res_vec
            fx = sx.astype(jnp.int32)
            fy = sy.astype(jnp.int32)
            fz = sz.astype(jnp.int32)
            w_v[0, pl.ds(o, 16)] = sx - fx.astype(jnp.float32)
            w_v[1, pl.ds(o, 16)] = sy - fy.astype(jnp.float32)
            w_v[2, pl.ds(o, 16)] = sz - fz.astype(jnp.float32)
            b0 = fy * _K2
            c0 = fz * _K3
            ax = (fx, fx + jnp.int32(1))
            by = (b0, b0 + _K2)
            cz = (c0, c0 + _K3)
            for zb in range(2):
                for yb in range(2):
                    for xb in range(2):
                        h = ((ax[xb] ^ by[yb] ^ cz[zb]) & _MASK) + loff
                        slot = xb + 2 * yb + 4 * zb
                        idx_v[pl.ds(jnp.int32(slot * _P) + o, 16)] = h


    def interp_pass(l, rows_v, w_v):
        lf = l * jnp.int32(_F)
        cols = [jnp.full((16,), f, jnp.int32) for f in range(_F)]

        @plsc.parallel_loop(jnp.int32(0), jnp.int32(_G), jnp.int32(1),
                            unroll=4)
        def interp_body(g):
            o = g * jnp.int32(16)
            wx = w_v[0, pl.ds(o, 16)]
            wy = w_v[1, pl.ds(o, 16)]
            wz = w_v[2, pl.ds(o, 16)]
            ux = 1.0 - wx
            uy = 1.0 - wy
            uz = 1.0 - wz
            p00 = ux * uy
            p10 = wx * uy
            p01 = ux * wy
            p11 = wx * wy
            w8 = [p00 * uz, p10 * uz, p01 * uz, p11 * uz,
                  p00 * wz, p10 * wz, p01 * wz, p11 * wz]
            r = o + lanes
            rows = [r + jnp.int32(s * _P) for s in range(8)]
            accs = [None] * _F
            for s in range(8):
                for f in range(_F):
                    v = plsc.load_gather(rows_v, [rows[s], cols[f]])
                    t = v * w8[s]
                    accs[f] = t if accs[f] is None else accs[f] + t
            for f in range(_F):
                ocol = jnp.broadcast_to(lf + jnp.int32(f), (16,))
                plsc.store_scatter(out_v, [r, ocol], accs[f])

    def start_gather(src, idx_v, rows_v, sa, sb):
        pltpu.async_copy(src.at[idx_v.at[pl.ds(0, _H)]],
                         rows_v.at[pl.ds(0, _H)], sa)
        pltpu.async_copy(src.at[idx_v.at[pl.ds(_H, _H)]],
                         rows_v.at[pl.ds(_H, _H)], sb)

    def wait_gather(src, idx_v, rows_v, sa, sb):
        pltpu.make_async_copy(src.at[idx_v.at[pl.ds(0, _H)]],
                              rows_v.at[pl.ds(0, _H)], sa).wait()
        pltpu.make_async_copy(src.at[idx_v.at[pl.ds(_H, _H)]],
                              rows_v.at[pl.ds(_H, _H)], sb).wait()

    def chunk_body(ci, carry):
        base = wid * jnp.int32(_PW) + ci * jnp.int32(_P)
        pltpu.async_copy(xyz.at[:, pl.ds(base, _P)], xyz_v, sem_in).wait()

        hash_pass(jnp.int32(15), jnp.int32(_T), idxs0_v, ws0_v)
        start_gather(spm_v, idxs0_v, rs0_v, sems0a, sems0b)

        hash_pass(jnp.int32(0), jnp.int32(0), idxh0_v, wh0_v)
        start_gather(table, idxh0_v, rh0_v, semh0a, semh0b)

        def dbl_body(k, carry2):
            j0 = k * jnp.int32(2)
            j1 = j0 + jnp.int32(1)
            hash_pass(j1, j1 * jnp.int32(_T), idxh1_v, wh1_v)
            start_gather(table, idxh1_v, rh1_v, semh1a, semh1b)

            wait_gather(table, idxh0_v, rh0_v, semh0a, semh0b)
            interp_pass(j0, rh0_v, wh0_v)

            j2 = j0 + jnp.int32(2)

            @pl.when(k < jnp.int32(6))
            def _prefetch_hbm():
                hash_pass(j2, j2 * jnp.int32(_T), idxh0_v, wh0_v)
                start_gather(table, idxh0_v, rh0_v, semh0a, semh0b)

            @pl.when(k == jnp.int32(6))
            def _prefetch_spm():
                hash_pass(j2, jnp.int32(0), idxh0_v, wh0_v)
                start_gather(spm_v, idxh0_v, rh0_v, semh0a, semh0b)

            wait_gather(table, idxh1_v, rh1_v, semh1a, semh1b)
            interp_pass(j1, rh1_v, wh1_v)
            return carry2

        lax.fori_loop(jnp.int32(0), jnp.int32(7), dbl_body, jnp.int32(0))

        wait_gather(table, idxh0_v, rh0_v, semh0a, semh0b)
        interp_pass(jnp.int32(14), rh0_v, wh0_v)
        wait_gather(spm_v, idxs0_v, rs0_v, sems0a, sems0b)
        interp_pass(jnp.int32(15), rs0_v, ws0_v)
        pltpu.async_copy(out_v, out.at[pl.ds(base, _P)], sem_out).wait()
        return carry

    lax.fori_loop(jnp.int32(0), jnp.int32(_NCHUNK), chunk_body, jnp.int32(0))


_hash_enc = functools.partial(
    pl.kernel,
    out_type=jax.ShapeDtypeStruct((_N, _NUM_LEVELS * _F), jnp.float32),
    mesh=plsc.VectorSubcoreMesh(core_axis_name="c", subcore_axis_name="s"),
    scratch_types=[
        pltpu.VMEM((3, _P), jnp.float32),
        pltpu.VMEM((16 * 16,), jnp.float32),
        pltpu.VMEM((3, _P), jnp.float32),
        pltpu.VMEM((3, _P), jnp.float32),
        pltpu.VMEM((3, _P), jnp.float32),
        pltpu.VMEM((8 * _P,), jnp.int32),
        pltpu.VMEM((8 * _P,), jnp.int32),
        pltpu.VMEM((8 * _P,), jnp.int32),
        pltpu.VMEM((8 * _P, _F), jnp.float32),
        pltpu.VMEM((8 * _P, _F), jnp.float32),
        pltpu.VMEM((8 * _P, _F), jnp.float32),
        pltpu.VMEM((_P, _NUM_LEVELS * _F), jnp.float32),
        pltpu.VMEM_SHARED((_SPM_ROWS, _F), jnp.float32),
        pltpu.SemaphoreType.DMA,
        pltpu.SemaphoreType.DMA,
        pltpu.SemaphoreType.DMA,
        pltpu.SemaphoreType.DMA,
        pltpu.SemaphoreType.DMA,
        pltpu.SemaphoreType.DMA,
        pltpu.SemaphoreType.DMA,
        pltpu.SemaphoreType.DMA,
    ],
    compiler_params=pltpu.CompilerParams(
        needs_layout_passes=False, use_tc_tiling_on_sc=False),
)(_body)


def kernel(inp_points, hash_table):
    xyz = inp_points.T
    resb = jnp.asarray(np.repeat(_RES, 16))
    return _hash_enc(xyz, hash_table, resb)


# speedup vs baseline: 1.2010x; 1.0014x over previous
import functools

import numpy as np
import jax
import jax.numpy as jnp
from jax import lax
from jax.experimental import pallas as pl
from jax.experimental.pallas import tpu as pltpu
from jax.experimental.pallas import tpu_sc as plsc

_NUM_LEVELS = 16
_LOG2_T = 14
_T = 1 << _LOG2_T
_F = 8
_N = 131072
_GROWTH = np.exp((np.log(1024.0) - np.log(16.0)) / (_NUM_LEVELS - 1))
_RES = np.floor(16.0 * _GROWTH ** np.arange(_NUM_LEVELS)).astype(np.float32)

_K2 = np.uint32(2654435761).astype(np.int32)
_K3 = np.int32(805459861)
_MASK = np.int32(_T - 1)

_NC, _NS = 2, 16
_NW = _NC * _NS
_PW = _N // _NW
_P = 256
_NCHUNK = _PW // _P
_G = _P // 16
_H = 4 * _P
_NSPM = 2
_LSPM = _NUM_LEVELS - _NSPM
_SPM_ROWS = _NSPM * _T
_CP = _SPM_ROWS // _NS


def _body(xyz, table, resb, out, xyz_v, res_v, wh0_v, wh1_v, ws0_v,
          idxh0_v, idxh1_v, idxs0_v, rh0_v, rh1_v, rs0_v,
          out_v, spm_v, sem_in, semh0a, semh0b, semh1a, semh1b, sems0a,
          sems0b, sem_out):
    wid = lax.axis_index("s") * jnp.int32(_NC) + lax.axis_index("c")
    sid = lax.axis_index("s")
    lanes = lax.iota(jnp.int32, 16)

    pltpu.async_copy(resb, res_v, sem_in).wait()

    cbase = sid * jnp.int32(_CP)
    pltpu.async_copy(
        table.at[pl.ds(jnp.int32(_LSPM * _T) + cbase, _CP)],
        spm_v.at[pl.ds(cbase, _CP)], sem_in).wait()
    plsc.subcore_barrier()

    def hash_pass(l, loff, idx_v, w_v):
        res_vec = res_v[pl.ds(l * jnp.int32(16), 16)]

        @plsc.parallel_loop(jnp.int32(0), jnp.int32(_G), jnp.int32(1),
                            unroll=4)
        def hash_body(g):
            o = g * jnp.int32(16)
            x = xyz_v[0, pl.ds(o, 16)]
            y = xyz_v[1, pl.ds(o, 16)]
            z = xyz_v[2, pl.ds(o, 16)]
            sx = x * res_vec
            sy = y * res_vec
            sz = z * res_vec
            fx = sx.astype(jnp.int32)
            fy = sy.astype(jnp.int32)
            fz = sz.astype(jnp.int32)
            w_v[0, pl.ds(o, 16)] = sx - fx.astype(jnp.float32)
            w_v[1, pl.ds(o, 16)] = sy - fy.astype(jnp.float32)
            w_v[2, pl.ds(o, 16)] = sz - fz.astype(jnp.float32)
            b0 = fy * _K2
            c0 = fz * _K3
            ax = (fx, fx + jnp.int32(1))
            by = (b0, b0 + _K2)
            cz = (c0, c0 + _K3)
            for zb in range(2):
                for yb in range(2):
                    for xb in range(2):
                        h = ((ax[xb] ^ by[yb] ^ cz[zb]) & _MASK) + loff
                        slot = xb + 2 * yb + 4 * zb
                        idx_v[pl.ds(jnp.int32(slot * _P) + o, 16)] = h


    def interp_pass(l, rows_v, w_v):
        lf = l * jnp.int32(_F)
        cols = [jnp.full((16,), f, jnp.int32) for f in range(_F)]

        @plsc.parallel_loop(jnp.int32(0), jnp.int32(_G), jnp.int32(1),
                            unroll=4)
        def interp_body(g):
            o = g * jnp.int32(16)
            wx = w_v[0, pl.ds(o, 16)]
            wy = w_v[1, pl.ds(o, 16)]
            wz = w_v[2, pl.ds(o, 16)]
            ux = 1.0 - wx
            uy = 1.0 - wy
            uz = 1.0 - wz
            p00 = ux * uy
            p10 = wx * uy
            p01 = ux * wy
            p11 = wx * wy
            w8 = [p00 * uz, p10 * uz, p01 * uz, p11 * uz,
                  p00 * wz, p10 * wz, p01 * wz, p11 * wz]
            r = o + lanes
            rows = [r + jnp.int32(s * _P) for s in range(8)]
            accs = [None] * _F
            for s in range(8):
                for f in range(_F):
                    v = plsc.load_gather(rows_v, [rows[s], cols[f]])
                    t = v * w8[s]
                    accs[f] = t if accs[f] is None else accs[f] + t
            for f in range(_F):
                ocol = jnp.broadcast_to(lf + jnp.int32(f), (16,))
                plsc.store_scatter(out_v, [r, ocol], accs[f])

    def start_gather(src, idx_v, rows_v, sa, sb):
        pltpu.async_copy(src.at[idx_v.at[pl.ds(0, _H)]],
                         rows_v.at[pl.ds(0, _H)], sa)
        pltpu.async_copy(src.at[idx_v.at[pl.ds(_H, _H)]],
                         rows_v.at[pl.ds(_H, _H)], sb)

    def wait_gather(src, idx_v, rows_v, sa, sb):
        pltpu.make_async_copy(src.at[idx_v.at[pl.ds(0, _H)]],
                              rows_v.at[pl.ds(0, _H)], sa).wait()
        pltpu.make_async_copy(src.at[idx_v.at[pl.ds(_H, _H)]],
                              rows_v.at[pl.ds(_H, _H)], sb).wait()

    def chunk_body(ci, carry):
        base = wid * jnp.int32(_PW) + ci * jnp.int32(_P)
        pltpu.async_copy(xyz.at[:, pl.ds(base, _P)], xyz_v, sem_in).wait()

        hash_pass(jnp.int32(15), jnp.int32(_T), idxs0_v, ws0_v)
        start_gather(spm_v, idxs0_v, rs0_v, sems0a, sems0b)

        hash_pass(jnp.int32(0), jnp.int32(0), idxh0_v, wh0_v)
        start_gather(table, idxh0_v, rh0_v, semh0a, semh0b)

        def dbl_body(k, carry2):
            j0 = k * jnp.int32(2)
            j1 = j0 + jnp.int32(1)
            hash_pass(j1, j1 * jnp.int32(_T), idxh1_v, wh1_v)
            start_gather(table, idxh1_v, rh1_v, semh1a, semh1b)

            wait_gather(table, idxh0_v, rh0_v, semh0a, semh0b)
            interp_pass(j0, rh0_v, wh0_v)

            j2 = j0 + jnp.int32(2)

            @pl.when(k < jnp.int32(6))
            def _prefetch_hbm():
                hash_pass(j2, j2 * jnp.int32(_T), idxh0_v, wh0_v)
                start_gather(table, idxh0_v, rh0_v, semh0a, semh0b)

            @pl.when(k == jnp.int32(6))
            def _prefetch_spm():
                hash_pass(j2, jnp.int32(0), idxh0_v, wh0_v)
                start_gather(spm_v, idxh0_v, rh0_v, semh0a, semh0b)

            wait_gather(table, idxh1_v, rh1_v, semh1a, semh1b)
            interp_pass(j1, rh1_v, wh1_v)
            return carry2

        lax.fori_loop(jnp.int32(0), jnp.int32(7), dbl_body, jnp.int32(0))

        wait_gather(table, idxh0_v, rh0_v, semh0a, semh0b)
        interp_pass(jnp.int32(14), rh0_v, wh0_v)
        wait_gather(spm_v, idxs0_v, rs0_v, sems0a, sems0b)
        interp_pass(jnp.int32(15), rs0_v, ws0_v)
        pltpu.async_copy(out_v, out.at[pl.ds(base, _P)], sem_out).wait()
        return carry

    lax.fori_loop(jnp.int32(0), jnp.int32(_NCHUNK), chunk_body, jnp.int32(0))


_hash_enc = functools.partial(
    pl.kernel,
    out_type=jax.ShapeDtypeStruct((_N, _NUM_LEVELS * _F), jnp.float32),
    mesh=plsc.VectorSubcoreMesh(core_axis_name="c", subcore_axis_name="s"),
    scratch_types=[
        pltpu.VMEM((3, _P), jnp.float32),
        pltpu.VMEM((16 * 16,), jnp.float32),
        pltpu.VMEM((3, _P), jnp.float32),
        pltpu.VMEM((3, _P), jnp.float32),
        pltpu.VMEM((3, _P), jnp.float32),
        pltpu.VMEM((8 * _P,), jnp.int32),
        pltpu.VMEM((8 * _P,), jnp.int32),
        pltpu.VMEM((8 * _P,), jnp.int32),
        pltpu.VMEM((8 * _P, _F), jnp.float32),
        pltpu.VMEM((8 * _P, _F), jnp.float32),
        pltpu.VMEM((8 * _P, _F), jnp.float32),
        pltpu.VMEM((_P, _NUM_LEVELS * _F), jnp.float32),
        pltpu.VMEM_SHARED((_SPM_ROWS, _F), jnp.float32),
        pltpu.SemaphoreType.DMA,
        pltpu.SemaphoreType.DMA,
        pltpu.SemaphoreType.DMA,
        pltpu.SemaphoreType.DMA,
        pltpu.SemaphoreType.DMA,
        pltpu.SemaphoreType.DMA,
        pltpu.SemaphoreType.DMA,
        pltpu.SemaphoreType.DMA,
    ],
    compiler_params=pltpu.CompilerParams(
        needs_layout_passes=False, use_tc_tiling_on_sc=False),
)(_body)


def kernel(inp_points, hash_table):
    xyz = inp_points.T
    resb = jnp.asarray(np.repeat(_RES, 16))
    return _hash_enc(xyz, hash_table, resb)
